# baseline (speedup 1.0000x reference)
import functools

import jax
import jax.numpy as jnp
from jax import lax
from jax.experimental import pallas as pl
from jax.experimental.pallas import tpu as pltpu
from jax.experimental.pallas import tpu_sc as plsc

NC = 2
NS = 16
NW = NC * NS
LANES = 16


def _make_sc_call(B, L, V, D):
  assert D == 64 and B % NW == 0 and L == 50 and V % 2 == 0
  bpw = B // NW
  blk_items = 8
  blk = blk_items * L
  half_items = blk_items // 2
  half_rows = half_items * L
  nblk = bpw // blk_items
  kd = D // LANES
  group_offs = (0, 16, 32, L - LANES)

  mesh = plsc.VectorSubcoreMesh(core_axis_name="c", subcore_axis_name="s")

  @functools.partial(
      pl.kernel,
      out_type=jax.ShapeDtypeStruct((B, L), jnp.float32),
      mesh=mesh,
      compiler_params=pltpu.CompilerParams(
          needs_layout_passes=False, use_tc_tiling_on_sc=True),
      scratch_types=[
          pltpu.VMEM((bpw + LANES,), jnp.int32),
          pltpu.VMEM((bpw,), jnp.int32),
          pltpu.VMEM((2, blk_items, L), jnp.int32),
          pltpu.VMEM((2 * blk,), jnp.int32),
          pltpu.VMEM((2 * blk,), jnp.int32),
          pltpu.VMEM((bpw * D,), jnp.float32),
          pltpu.VMEM((2, half_rows, 2 * D), jnp.float32),
          pltpu.VMEM((2, blk_items, L), jnp.float32),
          pltpu.SemaphoreType.DMA,
          pltpu.SemaphoreType.DMA,
          pltpu.SemaphoreType.DMA,
          pltpu.SemaphoreType.DMA,
          pltpu.SemaphoreType.DMA,
          pltpu.SemaphoreType.DMA,
      ],
  )
  def sc_call(center_hbm, ctx_hbm, ev_hbm, eu_hbm, out_hbm,
              cidx_v, cidx2_v, rawidx_v, flatidx_v, hb_v, vrows_v, ubuf_v,
              outbuf_v, semg0, semg1, semi, semv, semo0, semo1):
    wid = lax.axis_index("s") * NC + lax.axis_index("c")
    item0 = wid * bpw

    lane_iota = lax.iota(jnp.int32, LANES)

    pltpu.sync_copy(center_hbm.at[pl.ds(item0, bpw)], cidx_v.at[pl.ds(0, bpw)])
    for q in range(bpw // LANES):
      cidx2_v[pl.ds(q * LANES, LANES)] = (
          lax.shift_right_logical(cidx_v[pl.ds(q * LANES, LANES)], 1))
    for q in range(bpw // 128):
      pltpu.async_copy(
          ev_hbm.at[cidx2_v.at[pl.ds(q * 128, 128)]],
          ubuf_v.at[0, pl.ds(0, 128)], semv).wait()

      def compact(r, _):
        item = q * 128 + r
        hoff = ((cidx_v[pl.ds(item, LANES)] & 1) * D)[0]
        for k in range(kd):
          vrows_v[pl.ds(item * D + k * LANES, LANES)] = (
              ubuf_v[0, r, pl.ds(hoff + k * LANES, LANES)])
        return 0

      lax.fori_loop(0, 128, compact, 0)

    def idx_stage(s, buf):
      return pltpu.make_async_copy(
          ctx_hbm.at[pl.ds(item0 + s * blk_items, blk_items), :],
          rawidx_v.at[buf], semi)

    def flatten_idx(parity):
      for il in range(blk_items):
        for off in group_offs:
          r = rawidx_v[parity, il, pl.ds(off, LANES)]
          dst = parity * blk + il * L + off
          flatidx_v[pl.ds(dst, LANES)] = lax.shift_right_logical(r, 1)
          hb_v[pl.ds(dst, LANES)] = r & 1

    def u_gather(sp, h, sem):
      return pltpu.make_async_copy(
          eu_hbm.at[flatidx_v.at[pl.ds(sp * blk + h * half_rows,
                                       half_rows)]],
          ubuf_v.at[h], sem)

    nsub = 2 * nblk

    idx_stage(0, 0).start()
    idx_stage(0, 0).wait()
    flatten_idx(0)
    idx_stage(1, 1).start()
    u_gather(0, 0, semg0).start()
    u_gather(0, 1, semg1).start()

    def compute_item(il, c, sp, h):
      item = c * half_items + il
      fbase = sp * blk + (h * half_items + il) * L
      vvec = [vrows_v[pl.ds(item * D + k * LANES, LANES)] for k in range(kd)]

      for off in group_offs:
        acc_lo = jnp.full((LANES,), 0.0, jnp.float32)
        acc_hi = jnp.full((LANES,), 0.0, jnp.float32)
        for j in range(LANES):
          row = il * L + off + j
          p_lo = ubuf_v[h, row, pl.ds(0, LANES)] * vvec[0]
          p_hi = ubuf_v[h, row, pl.ds(D, LANES)] * vvec[0]
          for k in range(1, kd):
            p_lo = p_lo + (
                ubuf_v[h, row, pl.ds(k * LANES, LANES)] * vvec[k])
            p_hi = p_hi + (
                ubuf_v[h, row, pl.ds(D + k * LANES, LANES)] * vvec[k])
          acc_lo = jnp.where(lane_iota == j,
                             lax.reduce_sum(p_lo, axes=(0,)), acc_lo)
          acc_hi = jnp.where(lane_iota == j,
                             lax.reduce_sum(p_hi, axes=(0,)), acc_hi)
        par = hb_v[pl.ds(fbase + off, LANES)]
        outbuf_v[sp, h * half_items + il, pl.ds(off, LANES)] = (
            jnp.where(par != 0, acc_hi, acc_lo))

    def out_copy(S, sp, sem):
      return pltpu.make_async_copy(
          outbuf_v.at[sp],
          out_hbm.at[pl.ds(item0 + S * blk_items, blk_items), :], sem)

    def quad_body(g, carry):
      for sq in range(4):
        sp, h = (sq // 2) % 2, sq % 2
        c = g * 4 + sq
        semg = semg0 if h == 0 else semg1
        semo = semo0 if sp == 0 else semo1
        u_gather(sp, h, semg).wait()

        if h == 0:
          @pl.when(c >= 4)
          def _(c=c, sp=sp, semo=semo):
            out_copy(c // 2 - 2, sp, semo).wait()

        lax.fori_loop(
            0, half_items,
            lambda il, _, c=c, sp=sp, h=h:
                (compute_item(il, c, sp, h), 0)[1], 0)

        if h == 0:
          @pl.when(c + 2 < nsub)
          def _(c=c, sp=sp):
            idx_stage(c // 2 + 1, 1 - sp).wait()
            flatten_idx(1 - sp)

            @pl.when(c + 4 < nsub)
            def _():
              idx_stage(c // 2 + 2, sp).start()

        @pl.when(c + 2 < nsub)
        def _(sp=sp, h=h, semg=semg):
          u_gather(1 - sp, h, semg).start()

        if h == 1:
          out_copy(c // 2, sp, semo).start()

      return carry

    lax.fori_loop(0, nsub // 4, quad_body, 0)

    out_copy(nblk - 2, 0, semo0).wait()
    out_copy(nblk - 1, 1, semo1).wait()

  return sc_call


def kernel(center, context_negative, embed_v, embed_u):
  B, L = context_negative.shape
  V, D = embed_u.shape
  ev2 = embed_v.reshape(V // 2, 2 * D)
  eu2 = embed_u.reshape(V // 2, 2 * D)
  sc_call = _make_sc_call(B, L, V, D)
  return sc_call(center, context_negative, ev2, eu2)


# speedup vs baseline: 1.1263x; 1.1263x over previous
import functools

import jax
import jax.numpy as jnp
from jax import lax
from jax.experimental import pallas as pl
from jax.experimental.pallas import tpu as pltpu
from jax.experimental.pallas import tpu_sc as plsc

NC = 2
NS = 16
NW = NC * NS
LANES = 16


def _make_sc_call(B, L, V, D):
  assert D == 64 and B % NW == 0 and L == 50
  bpw = B // NW
  chunk_items = 8
  rows = chunk_items * L
  nchunk = bpw // chunk_items
  kd = D // LANES
  group_offs = (0, 16, 32, L - LANES)

  mesh = plsc.VectorSubcoreMesh(core_axis_name="c", subcore_axis_name="s")

  @functools.partial(
      pl.kernel,
      out_type=jax.ShapeDtypeStruct((B, L), jnp.float32),
      mesh=mesh,
      compiler_params=pltpu.CompilerParams(
          needs_layout_passes=False, use_tc_tiling_on_sc=False),
      scratch_types=[
          pltpu.VMEM((bpw,), jnp.int32),
          pltpu.VMEM((2, chunk_items, L), jnp.int32),
          pltpu.VMEM((2 * rows,), jnp.int32),
          pltpu.VMEM((bpw, D), jnp.float32),
          pltpu.VMEM((2, rows, D), jnp.float32),
          pltpu.VMEM((bpw, L), jnp.float32),
          pltpu.SemaphoreType.DMA,
          pltpu.SemaphoreType.DMA,
          pltpu.SemaphoreType.DMA,
          pltpu.SemaphoreType.DMA,
          pltpu.SemaphoreType.DMA,
      ],
  )
  def sc_call(center_hbm, ctx_hbm, ev_hbm, eu_hbm, out_hbm,
              cidx_v, rawidx_v, flatidx_v, vrows_v, ubuf_v, outbuf_v,
              semg0, semg1, semi0, semi1, semv):
    wid = lax.axis_index("s") * NC + lax.axis_index("c")
    item0 = wid * bpw

    pltpu.sync_copy(center_hbm.at[pl.ds(item0, bpw)], cidx_v)
    pltpu.async_copy(ev_hbm.at[cidx_v], vrows_v, semv).wait()

    lane_iota = lax.iota(jnp.int32, LANES)

    def idx_stage(c, buf, sem):
      return pltpu.make_async_copy(
          ctx_hbm.at[pl.ds(item0 + c * chunk_items, chunk_items), :],
          rawidx_v.at[buf], sem)

    def flatten_idx(parity):
      for il in range(chunk_items):
        for off in group_offs:
          flatidx_v[pl.ds(parity * rows + il * L + off, LANES)] = (
              rawidx_v[parity, il, pl.ds(off, LANES)])

    def u_gather(buf, sem):
      return pltpu.make_async_copy(
          eu_hbm.at[flatidx_v.at[pl.ds(buf * rows, rows)]],
          ubuf_v.at[buf], sem)

    idx_stage(0, 0, semi0).start()
    idx_stage(1, 1, semi1).start()
    idx_stage(0, 0, semi0).wait()
    flatten_idx(0)
    u_gather(0, semg0).start()
    idx_stage(1, 1, semi1).wait()
    flatten_idx(1)
    u_gather(1, semg1).start()

    def compute_item(il, c, parity):
      item = c * chunk_items + il
      ub_row0 = il * L
      vvec = [vrows_v[item, pl.ds(k * LANES, LANES)] for k in range(kd)]

      def group(base):
        acc = jnp.full((LANES,), 0.0, jnp.float32)
        for j in range(LANES):
          p = ubuf_v[parity, base + j, pl.ds(0, LANES)] * vvec[0]
          for k in range(1, kd):
            p = p + ubuf_v[parity, base + j, pl.ds(k * LANES, LANES)] * vvec[k]
          acc = jnp.where(lane_iota == j, lax.reduce_sum(p, axes=(0,)), acc)
        return acc

      for off in group_offs:
        outbuf_v[item, pl.ds(off, LANES)] = group(ub_row0 + off)

    def chunk_body(c, parity, semg, semi):
      u_gather(parity, semg).wait()

      @pl.when(c + 2 < nchunk)
      def _():
        idx_stage(c + 2, parity, semi).start()

      lax.fori_loop(
          0, chunk_items, lambda il, _: (compute_item(il, c, parity), 0)[1], 0)

      @pl.when(c + 2 < nchunk)
      def _():
        idx_stage(c + 2, parity, semi).wait()
        flatten_idx(parity)
        u_gather(parity, semg).start()

    def pair_body(gidx, carry):
      chunk_body(gidx * 2, 0, semg0, semi0)
      chunk_body(gidx * 2 + 1, 1, semg1, semi1)
      return carry

    lax.fori_loop(0, nchunk // 2, pair_body, 0)

    pltpu.sync_copy(outbuf_v, out_hbm.at[pl.ds(item0, bpw), :])

  return sc_call


def kernel(center, context_negative, embed_v, embed_u):
  B, L = context_negative.shape
  V, D = embed_u.shape
  sc_call = _make_sc_call(B, L, V, D)
  return sc_call(center, context_negative, embed_v, embed_u)
